# Initial kernel scaffold; baseline (speedup 1.0000x reference)
import jax
import jax.numpy as jnp
from jax.experimental import pallas as pl


def kernel(x, durations, max_len):
    raise NotImplementedError("write your pallas kernel here")


# speedup vs baseline: 20.8205x; 20.8205x over previous
import functools

import jax
import jax.numpy as jnp
from jax import lax
from jax.experimental import pallas as pl
from jax.experimental.pallas import tpu as pltpu
from jax.experimental.pallas import tpu_sc as plsc

B, T, D, L = 16, 512, 1024, 2048
NW = 32
RPW = (B * L) // NW
CH = 64
NCH = RPW // CH


def _route_body(dur_ref, maxlen_ref, idx_ref, mel_ref, nvw_ref):
    b = pl.program_id(0)
    dur = jnp.maximum(dur_ref[0, 0, :], 0)
    total = jnp.sum(dur)
    rk = lax.broadcasted_iota(jnp.int32, (T, T), 0)
    ci = lax.broadcasted_iota(jnp.int32, (T, T), 1)
    cum = jnp.sum(jnp.where(rk <= ci, dur[:, None], 0), axis=0)
    pos = lax.broadcasted_iota(jnp.int32, (T, L), 1)
    idx = jnp.sum((cum[:, None] <= pos).astype(jnp.int32), axis=0)
    idx = jnp.minimum(idx, T - 1)
    idx_ref[0, 0, :] = b * T + idx
    mel_ref[b] = total
    vend = jnp.minimum(total, maxlen_ref[0])
    nv0 = jnp.clip(vend, 0, RPW)
    nv1 = jnp.clip(vend - RPW, 0, RPW)
    nvw_ref[2 * b] = nv0
    nvw_ref[2 * b + 1] = nv1


def _route(durations, max_len_arr):
    return pl.pallas_call(
        _route_body,
        grid=(B,),
        in_specs=[
            pl.BlockSpec((1, 1, T), lambda b: (b, 0, 0)),
            pl.BlockSpec(memory_space=pltpu.SMEM),
        ],
        out_specs=[
            pl.BlockSpec((1, 1, L), lambda b: (b, 0, 0)),
            pl.BlockSpec(memory_space=pltpu.SMEM),
            pl.BlockSpec(memory_space=pltpu.SMEM),
        ],
        out_shape=[
            jax.ShapeDtypeStruct((B, 1, L), jnp.int32),
            jax.ShapeDtypeStruct((B,), jnp.int32),
            jax.ShapeDtypeStruct((NW + 16,), jnp.int32),
        ],
    )(durations.reshape(B, 1, T), max_len_arr)


def _sc_gather_body(table_hbm, idx_hbm, nvw_hbm, out_hbm,
                    idx_v, rows_v, nvw_v, sem):
    cid = lax.axis_index("c")
    sid = lax.axis_index("s")
    wid = sid * 2 + cid
    row_base = wid * RPW
    pltpu.sync_copy(nvw_hbm, nvw_v)
    pltpu.sync_copy(idx_hbm.at[pl.ds(wid * NCH, NCH)], idx_v)
    nvalid = nvw_v[pl.ds(wid, 16)][0]
    zvec = jnp.zeros((16,), jnp.float32)
    for c in range(NCH):
        pltpu.async_copy(table_hbm.at[idx_v.at[c]], rows_v, sem).wait()
        nval_c = jnp.clip(nvalid - c * CH, 0, CH)

        def zero_row(r, carry):
            for k in range(D // 16):
                rows_v[r, pl.ds(k * 16, 16)] = zvec
            return carry

        lax.fori_loop(nval_c, CH, zero_row, 0)
        pltpu.sync_copy(rows_v, out_hbm.at[pl.ds(row_base + c * CH, CH)])


@functools.cache
def _sc_gather():
    return pl.kernel(
        _sc_gather_body,
        mesh=plsc.VectorSubcoreMesh(core_axis_name="c", subcore_axis_name="s"),
        out_type=jax.ShapeDtypeStruct((B * L, D), jnp.float32),
        scratch_types=[
            pltpu.VMEM((NCH, CH), jnp.int32),
            pltpu.VMEM((CH, D), jnp.float32),
            pltpu.VMEM((NW + 16,), jnp.int32),
            pltpu.SemaphoreType.DMA,
        ],
    )


def kernel(x, durations, max_len):
    max_len_arr = jnp.asarray(max_len, jnp.int32).reshape(1)
    idx, mel_lens, nvw = _route(durations, max_len_arr)
    table = x.reshape(B * T, D)
    idx2d = idx.reshape(NW * NCH, CH)
    out_flat = _sc_gather()(table, idx2d, nvw)
    return out_flat.reshape(B, L, D), mel_lens


# speedup vs baseline: 28.6035x; 1.3738x over previous
import functools

import jax
import jax.numpy as jnp
from jax import lax
from jax.experimental import pallas as pl
from jax.experimental.pallas import tpu as pltpu
from jax.experimental.pallas import tpu_sc as plsc

B, T, D, L = 16, 512, 1024, 2048
NW = 32
RPW = (B * L) // NW
CH = 32
NCH = RPW // CH


def _route_body(dur_ref, maxlen_ref, idx_ref, mel_ref, nvw_ref):
    b = pl.program_id(0)
    dur = jnp.maximum(dur_ref[0, 0, :], 0)
    total = jnp.sum(dur)
    rk = lax.broadcasted_iota(jnp.int32, (T, T), 0)
    ci = lax.broadcasted_iota(jnp.int32, (T, T), 1)
    cum = jnp.sum(jnp.where(rk <= ci, dur[:, None], 0), axis=0)
    pos = lax.broadcasted_iota(jnp.int32, (T, L), 1)
    idx = jnp.sum((cum[:, None] <= pos).astype(jnp.int32), axis=0)
    idx = jnp.minimum(idx, T - 1)
    idx_ref[0, 0, :] = b * T + idx
    mel_ref[b] = total
    vend = jnp.minimum(total, maxlen_ref[0])
    nv0 = jnp.clip(vend, 0, RPW)
    nv1 = jnp.clip(vend - RPW, 0, RPW)
    nvw_ref[2 * b] = nv0
    nvw_ref[2 * b + 1] = nv1


def _route(durations, max_len_arr):
    return pl.pallas_call(
        _route_body,
        grid=(B,),
        in_specs=[
            pl.BlockSpec((1, 1, T), lambda b: (b, 0, 0)),
            pl.BlockSpec(memory_space=pltpu.SMEM),
        ],
        out_specs=[
            pl.BlockSpec((1, 1, L), lambda b: (b, 0, 0)),
            pl.BlockSpec(memory_space=pltpu.SMEM),
            pl.BlockSpec(memory_space=pltpu.SMEM),
        ],
        out_shape=[
            jax.ShapeDtypeStruct((B, 1, L), jnp.int32),
            jax.ShapeDtypeStruct((B,), jnp.int32),
            jax.ShapeDtypeStruct((NW + 16,), jnp.int32),
        ],
    )(durations.reshape(B, 1, T), max_len_arr)


def _sc_gather_body(table_hbm, idx_hbm, nvw_hbm, out_hbm,
                    idx_v, buf0, buf1, zbuf, nvw_v,
                    gsem0, gsem1, ssem0, ssem1):
    cid = lax.axis_index("c")
    sid = lax.axis_index("s")
    wid = sid * 2 + cid
    row_base = wid * RPW
    pltpu.sync_copy(nvw_hbm, nvw_v)
    pltpu.sync_copy(idx_hbm.at[pl.ds(wid * NCH, NCH)], idx_v)
    nvalid = nvw_v[pl.ds(wid, 16)][0]
    zvec = jnp.zeros((16,), jnp.float32)
    bufs = (buf0, buf1)
    gsems = (gsem0, gsem1)
    ssems = (ssem0, ssem1)

    def zero_rows(buf, lo, hi):
        def zero_row(r, carry):
            for k in range(D // 16):
                buf[r, pl.ds(k * 16, 16)] = zvec
            return carry
        lax.fori_loop(lo, hi, zero_row, 0)

    zero_rows(zbuf, 0, CH)

    def nval(c):
        return jnp.clip(nvalid - c * CH, 0, CH)

    def start_gather(c):
        p = c % 2

        @pl.when(nval(c) > 0)
        def _():
            pltpu.async_copy(table_hbm.at[idx_v.at[c]], bufs[p], gsems[p])

    def wait_gather(c):
        p = c % 2

        @pl.when(nval(c) > 0)
        def _():
            pltpu.make_async_copy(
                table_hbm.at[idx_v.at[c]], bufs[p], gsems[p]).wait()

    def start_scatter(c):
        p = c % 2
        nv = nval(c)
        dst = out_hbm.at[pl.ds(row_base + c * CH, CH)]

        @pl.when(nv == 0)
        def _():
            pltpu.async_copy(zbuf, dst, ssems[p])

        @pl.when(nv > 0)
        def _():
            zero_rows(bufs[p], nv, CH)
            pltpu.async_copy(bufs[p], dst, ssems[p])

    def wait_scatter(c):
        p = c % 2
        pltpu.make_async_copy(
            zbuf, out_hbm.at[pl.ds(row_base + c * CH, CH)], ssems[p]).wait()

    start_gather(0)
    for c in range(NCH):
        if c + 1 < NCH:
            if c >= 1:
                wait_scatter(c - 1)
            start_gather(c + 1)
        wait_gather(c)
        start_scatter(c)
    wait_scatter(NCH - 2)
    wait_scatter(NCH - 1)


@functools.cache
def _sc_gather():
    return pl.kernel(
        _sc_gather_body,
        mesh=plsc.VectorSubcoreMesh(core_axis_name="c", subcore_axis_name="s"),
        out_type=jax.ShapeDtypeStruct((B * L, D), jnp.float32),
        scratch_types=[
            pltpu.VMEM((NCH, CH), jnp.int32),
            pltpu.VMEM((CH, D), jnp.float32),
            pltpu.VMEM((CH, D), jnp.float32),
            pltpu.VMEM((CH, D), jnp.float32),
            pltpu.VMEM((NW + 16,), jnp.int32),
            pltpu.SemaphoreType.DMA,
            pltpu.SemaphoreType.DMA,
            pltpu.SemaphoreType.DMA,
            pltpu.SemaphoreType.DMA,
        ],
    )


def kernel(x, durations, max_len):
    max_len_arr = jnp.asarray(max_len, jnp.int32).reshape(1)
    idx, mel_lens, nvw = _route(durations, max_len_arr)
    table = x.reshape(B * T, D)
    idx2d = idx.reshape(NW * NCH, CH)
    out_flat = _sc_gather()(table, idx2d, nvw)
    return out_flat.reshape(B, L, D), mel_lens


# speedup vs baseline: 30.3327x; 1.0605x over previous
import functools

import jax
import jax.numpy as jnp
from jax import lax
from jax.experimental import pallas as pl
from jax.experimental.pallas import tpu as pltpu
from jax.experimental.pallas import tpu_sc as plsc

B, T, D, L = 16, 512, 1024, 2048
NW = 32
RPW = (B * L) // NW
CH = 32
NCH = RPW // CH


def _route_body(dur_ref, maxlen_ref, idx_ref, mel_ref, nvw_ref):
    b = pl.program_id(0)
    dur = jnp.maximum(dur_ref[0, 0, :], 0)
    total = jnp.sum(dur)
    rk = lax.broadcasted_iota(jnp.int32, (T, T), 0)
    ci = lax.broadcasted_iota(jnp.int32, (T, T), 1)
    cum = jnp.sum(jnp.where(rk <= ci, dur[:, None], 0), axis=0)
    pos = lax.broadcasted_iota(jnp.int32, (T, L), 1)
    idx = jnp.sum((cum[:, None] <= pos).astype(jnp.int32), axis=0)
    idx = jnp.minimum(idx, T - 1)
    idx_ref[0, 0, :] = b * T + idx
    mel_ref[b] = total
    vend = jnp.minimum(total, maxlen_ref[0])
    nv0 = jnp.clip(vend, 0, RPW)
    nv1 = jnp.clip(vend - RPW, 0, RPW)
    nvw_ref[2 * b] = nv0
    nvw_ref[2 * b + 1] = nv1


def _route(durations, max_len_arr):
    return pl.pallas_call(
        _route_body,
        grid=(B,),
        in_specs=[
            pl.BlockSpec((1, 1, T), lambda b: (b, 0, 0)),
            pl.BlockSpec(memory_space=pltpu.SMEM),
        ],
        out_specs=[
            pl.BlockSpec((1, 1, L), lambda b: (b, 0, 0)),
            pl.BlockSpec(memory_space=pltpu.SMEM),
            pl.BlockSpec(memory_space=pltpu.SMEM),
        ],
        out_shape=[
            jax.ShapeDtypeStruct((B, 1, L), jnp.int32),
            jax.ShapeDtypeStruct((B,), jnp.int32),
            jax.ShapeDtypeStruct((NW + 16,), jnp.int32),
        ],
    )(durations.reshape(B, 1, T), max_len_arr)


def _sc_gather_body(table_hbm, idx_hbm, nvw_hbm, out_hbm,
                    idx_v, buf0, buf1, buf2, nvw_v,
                    gsem0, gsem1, gsem2, ssem0, ssem1, ssem2):
    cid = lax.axis_index("c")
    sid = lax.axis_index("s")
    wid = sid * 2 + cid
    row_base = wid * RPW
    pltpu.sync_copy(nvw_hbm, nvw_v)
    pltpu.sync_copy(idx_hbm.at[pl.ds(wid * RPW, RPW)], idx_v)
    nvalid = nvw_v[pl.ds(wid, 16)][0]
    zvec = jnp.zeros((16,), jnp.float32)
    bufs = (buf0, buf1, buf2)
    gsems = (gsem0, gsem1, gsem2)
    ssems = (ssem0, ssem1, ssem2)
    bufzero = [jnp.bool_(False)] * 3

    def zero_rows(buf, lo, hi):
        def zero_row(r, carry):
            for k in range(D // 16):
                buf[r, pl.ds(k * 16, 16)] = zvec
            return carry
        lax.fori_loop(lo, hi, zero_row, 0)

    def nval(c):
        return jnp.clip(nvalid - c * CH, 0, CH)

    def start_gather(c):
        p = c % 3

        @pl.when(nval(c) > 0)
        def _():
            pltpu.async_copy(
                table_hbm.at[idx_v.at[pl.ds(c * CH, CH)]], bufs[p], gsems[p])

    def wait_gather(c):
        p = c % 3

        @pl.when(nval(c) > 0)
        def _():
            pltpu.make_async_copy(
                table_hbm.at[idx_v.at[pl.ds(c * CH, CH)]],
                bufs[p], gsems[p]).wait()

    def fix_tail(c):
        p = c % 3
        nv = nval(c)
        lo = jnp.where(bufzero[p] & (nv == 0), CH, nv)
        zero_rows(bufs[p], lo, CH)
        bufzero[p] = nv == 0

    def start_scatter(c):
        p = c % 3
        pltpu.async_copy(
            bufs[p], out_hbm.at[pl.ds(row_base + c * CH, CH)], ssems[p])

    def wait_scatter(c):
        p = c % 3
        pltpu.make_async_copy(
            bufs[p], out_hbm.at[pl.ds(row_base + c * CH, CH)], ssems[p]).wait()

    start_gather(0)
    start_gather(1)
    for c in range(NCH):
        wait_gather(c)
        fix_tail(c)
        start_scatter(c)
        if c + 2 < NCH:
            if c >= 1:
                wait_scatter(c - 1)
            start_gather(c + 2)
    wait_scatter(NCH - 3)
    wait_scatter(NCH - 2)
    wait_scatter(NCH - 1)


@functools.cache
def _sc_gather():
    return pl.kernel(
        _sc_gather_body,
        mesh=plsc.VectorSubcoreMesh(core_axis_name="c", subcore_axis_name="s"),
        out_type=jax.ShapeDtypeStruct((B * L, D), jnp.float32),
        scratch_types=[
            pltpu.VMEM((RPW,), jnp.int32),
            pltpu.VMEM((CH, D), jnp.float32),
            pltpu.VMEM((CH, D), jnp.float32),
            pltpu.VMEM((CH, D), jnp.float32),
            pltpu.VMEM((NW + 16,), jnp.int32),
            pltpu.SemaphoreType.DMA,
            pltpu.SemaphoreType.DMA,
            pltpu.SemaphoreType.DMA,
            pltpu.SemaphoreType.DMA,
            pltpu.SemaphoreType.DMA,
            pltpu.SemaphoreType.DMA,
        ],
    )


def kernel(x, durations, max_len):
    max_len_arr = jnp.asarray(max_len, jnp.int32).reshape(1)
    idx, mel_lens, nvw = _route(durations, max_len_arr)
    table = x.reshape(B * T, D)
    idx_flat = idx.reshape(B * L)
    out_flat = _sc_gather()(table, idx_flat, nvw)
    return out_flat.reshape(B, L, D), mel_lens
